# Initial kernel scaffold; baseline (speedup 1.0000x reference)
import jax
import jax.numpy as jnp
from jax.experimental import pallas as pl


def kernel(pred_eps_x, target_eps_x, used_sigmas_x, pred_eps_h, eps_h, batch_ids, length_noise, length_used_sigmas, angle_noise, pred_param_noise):
    raise NotImplementedError("write your pallas kernel here")


# speedup vs baseline: 2.6584x; 2.6584x over previous
import functools

import jax
import jax.numpy as jnp
from jax import lax
from jax.experimental import pallas as pl
from jax.experimental.pallas import tpu as pltpu
from jax.experimental.pallas import tpu_sc as plsc

N = 262144
B = 4096
DH = 128

ROWS = 1024
NC = 2
NS = 16
NW = NC * NS
CHUNK = N // NW
CROWS = CHUNK // 128


def _contrib_body(px_ref, tx_ref, sg_ref, ph_ref, eh_ref, out_ref):
    sig2 = sg_ref[...] * sg_ref[...]
    dx = tx_ref[...] / sig2 - px_ref[...]
    cx = jnp.sum(0.5 * sig2 * dx * dx, axis=1, keepdims=True)
    dh = eh_ref[...] - ph_ref[...]
    ch = jnp.sum(dh * dh, axis=1, keepdims=True)
    out_ref[...] = cx + ch


def _per_atom_contrib(px, tx, sg, ph, eh):
    grid = N // ROWS
    return pl.pallas_call(
        _contrib_body,
        grid=(grid,),
        in_specs=[
            pl.BlockSpec((ROWS, 3), lambda i: (i, 0)),
            pl.BlockSpec((ROWS, 3), lambda i: (i, 0)),
            pl.BlockSpec((ROWS, 1), lambda i: (i, 0)),
            pl.BlockSpec((ROWS, DH), lambda i: (i, 0)),
            pl.BlockSpec((ROWS, DH), lambda i: (i, 0)),
        ],
        out_specs=pl.BlockSpec((ROWS, 1), lambda i: (i, 0)),
        out_shape=jax.ShapeDtypeStruct((N, 1), jnp.float32),
    )(px, tx, sg, ph, eh)


def _segsum_body(c_hbm, ids_hbm, ones_hbm, zeros_hbm, sums_hbm, cnts_hbm,
                 vals_v, idx_v, ones_v, ssum, scnt):
    cid = lax.axis_index("c")
    sid = lax.axis_index("s")
    wid = sid * NC + cid
    base = wid * CHUNK
    pltpu.sync_copy(c_hbm.at[pl.ds(base, CHUNK)], vals_v)
    pltpu.sync_copy(ids_hbm.at[pl.ds(base, CHUNK)], idx_v)
    pltpu.sync_copy(ones_hbm, ones_v)

    @pl.when(sid == 0)
    def _():
        pltpu.sync_copy(zeros_hbm, ssum)
        pltpu.sync_copy(zeros_hbm, scnt)

    plsc.subcore_barrier()
    pltpu.sync_copy(vals_v, ssum.at[idx_v], add=True)
    pltpu.sync_copy(ones_v, scnt.at[idx_v], add=True)
    plsc.subcore_barrier()

    @pl.when(sid == 0)
    def _():
        pltpu.sync_copy(ssum, sums_hbm.at[cid])
        pltpu.sync_copy(scnt, cnts_hbm.at[cid])


def _segsum(c2d, ids2d, ones2d, zeros):
    k = functools.partial(
        pl.kernel,
        mesh=plsc.VectorSubcoreMesh(core_axis_name="c", subcore_axis_name="s"),
        out_type=[
            jax.ShapeDtypeStruct((NC, B), jnp.float32),
            jax.ShapeDtypeStruct((NC, B), jnp.float32),
        ],
        scratch_types=[
            pltpu.VMEM((CHUNK,), jnp.float32),
            pltpu.VMEM((CHUNK,), jnp.int32),
            pltpu.VMEM((CHUNK,), jnp.float32),
            pltpu.VMEM_SHARED((B,), jnp.float32),
            pltpu.VMEM_SHARED((B,), jnp.float32),
        ],
    )(_segsum_body)
    return k(c2d, ids2d, ones2d, zeros)


def _final_body(s_ref, c_ref, ln_ref, ls_ref, an_ref, pp_ref, out_ref):
    s = jnp.sum(s_ref[...], axis=0)
    cnt = jnp.maximum(jnp.sum(c_ref[...], axis=0), 1.0)
    main = jnp.sum(s / cnt) * (1.0 / B)
    ls2 = ls_ref[...] * ls_ref[...]
    dl = pp_ref[:, 0:3] - ln_ref[...] / ls2
    length_loss = jnp.mean(0.5 * ls2 * dl * dl)
    da = pp_ref[:, 3:6] - an_ref[...]
    angle_loss = jnp.mean(da * da)
    out_ref[...] = jnp.full((1, 1), main + length_loss + angle_loss,
                            dtype=jnp.float32)


def _finalize(sums, cnts, ln, ls, an, pp):
    return pl.pallas_call(
        _final_body,
        out_shape=jax.ShapeDtypeStruct((1, 1), jnp.float32),
    )(sums, cnts, ln, ls, an, pp)


def kernel(pred_eps_x, target_eps_x, used_sigmas_x, pred_eps_h, eps_h,
           batch_ids, length_noise, length_used_sigmas, angle_noise,
           pred_param_noise):
    c = _per_atom_contrib(pred_eps_x, target_eps_x, used_sigmas_x,
                          pred_eps_h, eps_h)
    cflat = c.reshape(N)
    ids = batch_ids.astype(jnp.int32).reshape(N)
    ones = jnp.ones((CHUNK,), jnp.float32)
    zeros = jnp.zeros((B,), jnp.float32)
    sums, cnts = _segsum(cflat, ids, ones, zeros)
    out = _finalize(sums, cnts, length_noise, length_used_sigmas,
                    angle_noise, pred_param_noise)
    return out.reshape(())
